# Initial kernel scaffold; baseline (speedup 1.0000x reference)
import jax
import jax.numpy as jnp
from jax.experimental import pallas as pl


def kernel(hidden_states, position_ids, topk_mask, topk_scores, g1, g2, Wq, Wk, Wv, Wo, Wg, Wu, Wd):
    raise NotImplementedError("write your pallas kernel here")


# speedup vs baseline: 1.8978x; 1.8978x over previous
import functools

import numpy as np

import jax
import jax.numpy as jnp
from jax import lax
from jax.experimental import pallas as pl
from jax.experimental.pallas import tpu as pltpu
from jax.experimental.pallas import tpu_sc as plsc

_B, _S, _H, _NH, _HD, _F = 2, 4096, 1024, 16, 64, 2816
_EPS = 1e-5
_THETA = 10000.0
_BQ = 256
_BK = 512
_NQ = _S // _BQ
_TRASH = _B * _S
_NTILES = 32
_RPT = _B * _S // _NTILES
_SUB = 64

_INTERPRET = False


def _sc_index_build(mask_i32):
    mesh = plsc.VectorSubcoreMesh(core_axis_name="c", subcore_axis_name="s", num_cores=2, num_subcores=16)

    @functools.partial(
        pl.kernel,
        out_type=(
            jax.ShapeDtypeStruct((_B, _S), jnp.int32),
            jax.ShapeDtypeStruct((_B, 16), jnp.int32),
        ),
        mesh=mesh,
        scratch_types=[
            pltpu.VMEM((_S,), jnp.int32),
            pltpu.VMEM((_S,), jnp.int32),
            pltpu.VMEM((16,), jnp.int32),
        ],
        compiler_params=pltpu.CompilerParams(needs_layout_passes=False),
        interpret=_INTERPRET,
    )
    def k(mask_hbm, gidx_hbm, lens_hbm, mask_v, gidx_v, lens_v):
        wid = lax.axis_index("s") * 2 + lax.axis_index("c")

        @pl.when(wid == 0)
        def _():
            def batch_body(b, _):
                pltpu.sync_copy(mask_hbm.at[b], mask_v)
                neg1 = jnp.full((16,), -1, jnp.int32)

                def initb(i, c):
                    gidx_v[pl.ds(i * 16, 16)] = neg1
                    return c

                lax.fori_loop(0, _S // 16, initb, 0)
                base = b * _S

                def chunk(i, carry):
                    m = mask_v[pl.ds(i * 16, 16)]
                    mb = m != 0
                    c = plsc.cumsum(m)
                    rank = c - 1 + carry
                    tvec = lax.iota(jnp.int32, 16) + i * 16 + base
                    plsc.store_scatter(gidx_v, [rank], tvec, mask=mb)
                    return carry + jnp.sum(m)

                ln = lax.fori_loop(0, _S // 16, chunk, jnp.int32(0))
                pltpu.sync_copy(gidx_v, gidx_hbm.at[b])
                lens_v[...] = jnp.zeros((16,), jnp.int32) + ln
                pltpu.sync_copy(lens_v, lens_hbm.at[b])
                return 0

            lax.fori_loop(0, _B, batch_body, 0)

    return k(mask_i32)


def _sc_gather(hid_flat, gidx_flat):
    mesh = plsc.VectorSubcoreMesh(core_axis_name="c", subcore_axis_name="s", num_cores=2, num_subcores=16)

    @functools.partial(
        pl.kernel,
        out_type=jax.ShapeDtypeStruct((_B * _S, _H), jnp.float32),
        mesh=mesh,
        scratch_types=[
            pltpu.VMEM((_SUB,), jnp.int32),
            pltpu.VMEM((_SUB, _H), jnp.float32),
            pltpu.SemaphoreType.DMA,
        ],
        interpret=_INTERPRET,
    )
    def k(hid_hbm, gidx_hbm, out_hbm, idx_v, rows_v, sem):
        wid = lax.axis_index("s") * 2 + lax.axis_index("c")
        base = wid * _RPT
        bbase = (base // _S) * _S

        def sub(j, _):
            sb = base + j * _SUB
            pltpu.sync_copy(gidx_hbm.at[pl.ds(sb, _SUB)], idx_v)
            for t in range(_SUB // 16):
                g = idx_v[pl.ds(t * 16, 16)]
                idx_v[pl.ds(t * 16, 16)] = jnp.where(g < 0, bbase, g)
            pltpu.async_copy(hid_hbm.at[idx_v], rows_v, sem).wait()
            pltpu.sync_copy(rows_v, out_hbm.at[pl.ds(sb, _SUB)])
            return 0

        lax.fori_loop(0, _RPT // _SUB, sub, 0)

    return k(hid_flat, gidx_flat)


def _sc_scatter(hid_flat, lo_flat, mask_flat, gidx_flat):
    mesh = plsc.VectorSubcoreMesh(core_axis_name="c", subcore_axis_name="s", num_cores=2, num_subcores=16)

    @functools.partial(
        pl.kernel,
        out_type=jax.ShapeDtypeStruct((_B * _S + 8, _H), jnp.float32),
        mesh=mesh,
        scratch_types=[
            pltpu.VMEM((_SUB,), jnp.int32),
            pltpu.VMEM((_SUB,), jnp.int32),
            pltpu.VMEM((_SUB, _H), jnp.float32),
            pltpu.SemaphoreType.DMA,
        ],
        interpret=_INTERPRET,
    )
    def k(hid_hbm, lo_hbm, mask_hbm, gidx_hbm, out_hbm, m_v, idx_v, buf, sem):
        wid = lax.axis_index("s") * 2 + lax.axis_index("c")
        base = wid * _RPT

        def sub(j, _):
            sb = base + j * _SUB
            pltpu.sync_copy(mask_hbm.at[pl.ds(sb, _SUB)], m_v)
            for t in range(_SUB // 16):
                m = m_v[pl.ds(t * 16, 16)]
                tvec = lax.iota(jnp.int32, 16) + (sb + t * 16)
                idx_v[pl.ds(t * 16, 16)] = jnp.where(m != 0, _TRASH, tvec)
            pltpu.sync_copy(hid_hbm.at[pl.ds(sb, _SUB)], buf)
            pltpu.async_copy(buf, out_hbm.at[idx_v], sem).wait()
            pltpu.sync_copy(gidx_hbm.at[pl.ds(sb, _SUB)], m_v)
            for t in range(_SUB // 16):
                g = m_v[pl.ds(t * 16, 16)]
                idx_v[pl.ds(t * 16, 16)] = jnp.where(g < 0, _TRASH, g)
            pltpu.sync_copy(lo_hbm.at[pl.ds(sb, _SUB)], buf)
            pltpu.async_copy(buf, out_hbm.at[idx_v], sem).wait()
            return 0

        lax.fori_loop(0, _RPT // _SUB, sub, 0)

    return k(hid_flat, lo_flat, mask_flat, gidx_flat)


def _qkv_body(lens_ref, hs_ref, pos_ref, w_ref, g_ref, q_ref, k_ref, v_ref):
    b = pl.program_id(0)
    qi = pl.program_id(1)
    ln = lens_ref[b, 0]

    @pl.when(qi * _BQ < ln)
    def _():
        x = hs_ref[0]
        var = jnp.mean(x * x, axis=-1, keepdims=True)
        xn = (x * lax.rsqrt(var + _EPS)) * g_ref[0]
        qkv = jnp.dot(xn.astype(jnp.bfloat16), w_ref[...],
                      preferred_element_type=jnp.float32)
        pos = pos_ref[0].astype(jnp.float32) - b * float(_S)
        l_idx = lax.broadcasted_iota(jnp.int32, (1, _H), 1)
        jmod = (l_idx % 32).astype(jnp.float32)
        invf = jnp.exp(jmod * (-np.log(_THETA) / 32.0))
        ang = pos * invf
        c = jnp.cos(ang)
        s = jnp.sin(ang)
        sel = (l_idx % 64) < 32

        def rope(t):
            xp = jnp.concatenate([t[:, 32:], t[:, :32]], axis=1)
            xm = jnp.concatenate([t[:, -32:], t[:, :-32]], axis=1)
            return jnp.where(sel, -xp, xm)

        qp = qkv[:, :_H]
        kp = qkv[:, _H:2 * _H]
        q_ref[0] = (qp * c + rope(qp) * s).astype(jnp.bfloat16)
        k_ref[0] = (kp * c + rope(kp) * s).astype(jnp.bfloat16)
        v_ref[0] = qkv[:, 2 * _H:].astype(jnp.bfloat16)

    @pl.when(qi * _BQ >= ln)
    def _():
        z = jnp.zeros((_BQ, _H), jnp.bfloat16)
        q_ref[0] = z
        k_ref[0] = z
        v_ref[0] = z


def _qkv_call(lens_x, hs_c, pos3, wqkv, g1):
    grid_spec = pltpu.PrefetchScalarGridSpec(
        num_scalar_prefetch=1,
        grid=(_B, _NQ),
        in_specs=[
            pl.BlockSpec((1, _BQ, _H), lambda b, qi, L: (b, qi, 0)),
            pl.BlockSpec((1, _BQ, 1), lambda b, qi, L: (b * _NQ + qi, 0, 0)),
            pl.BlockSpec((_H, 3 * _H), lambda b, qi, L: (0, 0)),
            pl.BlockSpec((1, _H), lambda b, qi, L: (0, 0)),
        ],
        out_specs=[
            pl.BlockSpec((1, _BQ, _H), lambda b, qi, L: (b, qi, 0)),
            pl.BlockSpec((1, _BQ, _H), lambda b, qi, L: (b, qi, 0)),
            pl.BlockSpec((1, _BQ, _H), lambda b, qi, L: (b, qi, 0)),
        ],
    )
    shp = jax.ShapeDtypeStruct((_B, _S, _H), jnp.bfloat16)
    return pl.pallas_call(
        _qkv_body,
        grid_spec=grid_spec,
        out_shape=[shp, shp, shp],
        compiler_params=pltpu.CompilerParams(
            dimension_semantics=("parallel", "parallel")),
        interpret=_INTERPRET,
    )(lens_x, hs_c, pos3, wqkv, g1)


def _attn_body(lens_ref, q_ref, k_ref, v_ref, o_ref):
    b = pl.program_id(0)
    qi = pl.program_id(2)
    start = qi * _BQ
    ln = lens_ref[b, 0]

    @pl.when(start < ln)
    def _():
        qq = q_ref[0]
        row = start + lax.broadcasted_iota(jnp.int32, (_BQ, 1), 0)

        def kb_body(kb, carry):
            m0, l0, a0, m1, l1, a1 = carry
            kblk = k_ref[0, pl.ds(kb * _BK, _BK), :]
            vblk = v_ref[0, pl.ds(kb * _BK, _BK), :]
            col = kb * _BK + lax.broadcasted_iota(jnp.int32, (1, _BK), 1)
            ok = col <= row

            def one(q1, k1, v1, m, l, acc):
                s = lax.dot_general(q1, k1, (((1,), (1,)), ((), ())),
                                    preferred_element_type=jnp.float32)
                s = jnp.where(ok, s * (1.0 / np.sqrt(_HD)), -1e30)
                m_new = jnp.maximum(m, jnp.max(s, axis=1, keepdims=True))
                alpha = jnp.exp(m - m_new)
                p = jnp.exp(s - m_new)
                l_new = l * alpha + jnp.sum(p, axis=1, keepdims=True)
                acc_new = acc * alpha + jnp.dot(
                    p.astype(jnp.bfloat16), v1,
                    preferred_element_type=jnp.float32)
                return m_new, l_new, acc_new

            m0, l0, a0 = one(qq[:, :_HD], kblk[:, :_HD], vblk[:, :_HD],
                             m0, l0, a0)
            m1, l1, a1 = one(qq[:, _HD:], kblk[:, _HD:], vblk[:, _HD:],
                             m1, l1, a1)
            return m0, l0, a0, m1, l1, a1

        nkb = start // _BK + 1
        mi = jnp.full((_BQ, 1), -1e30, jnp.float32)
        li = jnp.zeros((_BQ, 1), jnp.float32)
        ai = jnp.zeros((_BQ, _HD), jnp.float32)
        m0, l0, a0, m1, l1, a1 = lax.fori_loop(
            0, nkb, kb_body, (mi, li, ai, mi, li, ai))
        o_ref[0] = jnp.concatenate(
            [(a0 / l0), (a1 / l1)], axis=1).astype(jnp.bfloat16)


def _attn_call(lens_x, q, k, v):
    grid_spec = pltpu.PrefetchScalarGridSpec(
        num_scalar_prefetch=1,
        grid=(_B, _NH // 2, _NQ),
        in_specs=[
            pl.BlockSpec((1, _BQ, 2 * _HD), lambda b, h, qi, L: (b, qi, h)),
            pl.BlockSpec((1, _S, 2 * _HD), lambda b, h, qi, L: (b, 0, h)),
            pl.BlockSpec((1, _S, 2 * _HD), lambda b, h, qi, L: (b, 0, h)),
        ],
        out_specs=pl.BlockSpec((1, _BQ, 2 * _HD),
                               lambda b, h, qi, L: (b, qi, h)),
    )
    return pl.pallas_call(
        _attn_body,
        grid_spec=grid_spec,
        out_shape=jax.ShapeDtypeStruct((_B, _S, _H), jnp.bfloat16),
        compiler_params=pltpu.CompilerParams(
            dimension_semantics=("parallel", "parallel", "arbitrary")),
        interpret=_INTERPRET,
    )(lens_x, q, k, v)


def _mlp_body(lens_ref, a_ref, hs_ref, wo_ref, g2_ref, wg_ref, wu_ref, wd_ref,
              o_ref):
    b = pl.program_id(0)
    qi = pl.program_id(1)
    ln = lens_ref[b, 0]

    @pl.when(qi * _BQ < ln)
    def _():
        r2 = hs_ref[0] + jnp.dot(a_ref[0], wo_ref[...],
                                 preferred_element_type=jnp.float32)
        var = jnp.mean(r2 * r2, axis=-1, keepdims=True)
        xn = ((r2 * lax.rsqrt(var + _EPS)) * g2_ref[0]).astype(jnp.bfloat16)
        g = jnp.dot(xn, wg_ref[...], preferred_element_type=jnp.float32)
        u = jnp.dot(xn, wu_ref[...], preferred_element_type=jnp.float32)
        act = (g * jax.nn.sigmoid(g) * u).astype(jnp.bfloat16)
        o_ref[0] = r2 + jnp.dot(act, wd_ref[...],
                                preferred_element_type=jnp.float32)


def _mlp_call(lens_x, attn, hs_c, wo, g2, wg, wu, wd):
    grid_spec = pltpu.PrefetchScalarGridSpec(
        num_scalar_prefetch=1,
        grid=(_B, _NQ),
        in_specs=[
            pl.BlockSpec((1, _BQ, _H), lambda b, qi, L: (b, qi, 0)),
            pl.BlockSpec((1, _BQ, _H), lambda b, qi, L: (b, qi, 0)),
            pl.BlockSpec((_H, _H), lambda b, qi, L: (0, 0)),
            pl.BlockSpec((1, _H), lambda b, qi, L: (0, 0)),
            pl.BlockSpec((_H, _F), lambda b, qi, L: (0, 0)),
            pl.BlockSpec((_H, _F), lambda b, qi, L: (0, 0)),
            pl.BlockSpec((_F, _H), lambda b, qi, L: (0, 0)),
        ],
        out_specs=pl.BlockSpec((1, _BQ, _H), lambda b, qi, L: (b, qi, 0)),
    )
    return pl.pallas_call(
        _mlp_body,
        grid_spec=grid_spec,
        out_shape=jax.ShapeDtypeStruct((_B, _S, _H), jnp.float32),
        compiler_params=pltpu.CompilerParams(
            dimension_semantics=("parallel", "parallel")),
        interpret=_INTERPRET,
    )(lens_x, attn, hs_c, wo, g2, wg, wu, wd)


def kernel(hidden_states, position_ids, topk_mask, topk_scores, g1, g2,
           Wq, Wk, Wv, Wo, Wg, Wu, Wd):
    mask_i = topk_mask.astype(jnp.int32)
    gidx, lens_x = _sc_index_build(mask_i)

    hid_flat = hidden_states.reshape(_B * _S, _H)
    hs_c_flat = _sc_gather(hid_flat, gidx.reshape(-1))
    hs_c = hs_c_flat.reshape(_B, _S, _H)

    pos3 = gidx.reshape(_B * _NQ, _BQ, 1)
    wqkv = jnp.concatenate([Wq, Wk, Wv], axis=1).astype(jnp.bfloat16)
    q, k, v = _qkv_call(lens_x, hs_c, pos3, wqkv, g1.reshape(1, _H))

    attn = _attn_call(lens_x, q, k, v)

    layer_out = _mlp_call(lens_x, attn, hs_c,
                          Wo.astype(jnp.bfloat16), g2.reshape(1, _H),
                          Wg.astype(jnp.bfloat16), Wu.astype(jnp.bfloat16),
                          Wd.astype(jnp.bfloat16))

    outp = _sc_scatter(hid_flat, layer_out.reshape(_B * _S, _H),
                       mask_i.reshape(-1), gidx.reshape(-1))
    return outp[:_B * _S].reshape(_B, _S, _H)
